# Initial kernel scaffold; baseline (speedup 1.0000x reference)
import jax
import jax.numpy as jnp
from jax.experimental import pallas as pl


def kernel(indices, table):
    raise NotImplementedError("write your pallas kernel here")


# speedup vs baseline: 2.9605x; 2.9605x over previous
import functools

import jax
import jax.numpy as jnp
from jax import lax
from jax.experimental import pallas as pl
from jax.experimental.pallas import tpu as pltpu
from jax.experimental.pallas import tpu_sc as plsc

_D = 32
_B = 16384
_H = 50
_NC = 2
_NS = 16
_NW = _NC * _NS
_R = _B // _NW
_G = 32
_NG = _R // _G
_CH = 100
_C = (_G * _H) // _CH
_ROWS = _G * _H


def _sc_body(idx_hbm, table_hbm, out_hbm, idxv, bufv, outv, isem, gsem, osem):
    wid = lax.axis_index("s") * _NC + lax.axis_index("c")

    def idx_copy(g, slot):
        return pltpu.make_async_copy(
            idx_hbm.at[wid, g], idxv.at[pl.ds(slot * _C, _C)], isem)

    def gather_copy(c, slot):
        return pltpu.make_async_copy(
            table_hbm.at[idxv.at[slot * _C + c]],
            bufv.at[pl.ds(slot * _ROWS + c * _CH, _CH)], gsem)

    def out_copy(g):
        return pltpu.make_async_copy(
            outv, out_hbm.at[pl.ds(wid * _R + g * _G, _G)], osem)

    idx_copy(0, 0).start()
    idx_copy(0, 0).wait()
    for c in range(_C):
        gather_copy(c, 0).start()
    idx_copy(1, 1).start()

    def group_body(g, carry):
        p = lax.rem(g, 2)
        pn = lax.rem(g + 1, 2)

        for c in range(_C):
            gather_copy(c, p).wait()

        @pl.when(g + 1 < _NG)
        def _():
            idx_copy(g + 1, pn).wait()
            for c in range(_C):
                gather_copy(c, pn).start()

        @pl.when(g + 2 < _NG)
        def _():
            idx_copy(g + 2, p).start()

        @pl.when(g > 0)
        def _():
            out_copy(g - 1).wait()

        base = p * _ROWS

        def acc_body(i, c2):
            r0 = base + i * _H
            lo = [bufv[r0 + k, 0:16] for k in range(4)]
            hi = [bufv[r0 + k, 16:32] for k in range(4)]
            for j in range(4, _H):
                k = j % 4
                lo[k] = lo[k] + bufv[r0 + j, 0:16]
                hi[k] = hi[k] + bufv[r0 + j, 16:32]
            outv[i, 0:16] = (lo[0] + lo[1]) + (lo[2] + lo[3])
            outv[i, 16:32] = (hi[0] + hi[1]) + (hi[2] + hi[3])
            return c2

        lax.fori_loop(0, _G, acc_body, 0)
        out_copy(g).start()
        return carry

    lax.fori_loop(0, _NG, group_body, 0)
    out_copy(_NG - 1).wait()


@jax.jit
def kernel(indices, table):
    idx4 = indices.reshape(_NW, _NG, _C, _CH)
    f = pl.kernel(
        _sc_body,
        out_type=jax.ShapeDtypeStruct((_B, _D), jnp.float32),
        mesh=plsc.VectorSubcoreMesh(core_axis_name="c", subcore_axis_name="s"),
        scratch_types=[
            pltpu.VMEM((2 * _C, _CH), jnp.int32),
            pltpu.VMEM((2 * _ROWS, _D), jnp.float32),
            pltpu.VMEM((_G, _D), jnp.float32),
            pltpu.SemaphoreType.DMA,
            pltpu.SemaphoreType.DMA,
            pltpu.SemaphoreType.DMA,
        ],
        compiler_params=pltpu.CompilerParams(use_tc_tiling_on_sc=False),
    )
    return f(idx4, table)
